# Initial kernel scaffold; baseline (speedup 1.0000x reference)
import jax
import jax.numpy as jnp
from jax.experimental import pallas as pl


def kernel(x, edge_index, edge_attr, Ws1, Wn1, bs1, Wi, bi, Wf, bf, Wg, bg, Wo, bo, Ws2, Wn2, bs2, Wl1, bl1, Wl2, bl2):
    raise NotImplementedError("write your pallas kernel here")


# speedup vs baseline: 9.4241x; 9.4241x over previous
import functools

import jax
import jax.numpy as jnp
from jax import lax
from jax.experimental import pallas as pl
from jax.experimental.pallas import tpu as pltpu
from jax.experimental.pallas import tpu_sc as plsc

N = 50000
E = 800000
D = 32

NC = 2
NS = 16
NW = NC * NS

EB = 128
SUB = 4
CHUNK = EB * SUB
E_PAD = 819200
PER_W = E_PAD // NW
N_ITER = PER_W // CHUNK

N_PAD = 50176
STRIPE = N_PAD // NS

_mesh = plsc.VectorSubcoreMesh(core_axis_name="c", subcore_axis_name="s")
_sc_params = pltpu.CompilerParams(use_tc_tiling_on_sc=False)


@functools.partial(
    pl.kernel,
    out_type=jax.ShapeDtypeStruct((NC, N_PAD, D), jnp.float32),
    mesh=_mesh,
    compiler_params=_sc_params,
    scratch_types=[
        pltpu.VMEM_SHARED((N_PAD, D), jnp.float32),
        pltpu.VMEM((SUB, EB), jnp.int32),
        pltpu.VMEM((SUB, EB), jnp.int32),
        pltpu.VMEM((SUB, EB), jnp.float32),
        pltpu.VMEM((SUB, EB, D), jnp.float32),
        pltpu.SemaphoreType.DMA,
        pltpu.SemaphoreType.DMA,
    ],
)
def _sc_pass1(q_hbm, src_hbm, dst_hbm, ea_hbm, zero_hbm, out_hbm,
              acc, srcb, dstb, eab, rows, gsem, ssem):
    cid = lax.axis_index("c")
    sid = lax.axis_index("s")
    wid = cid * NS + sid

    pltpu.sync_copy(zero_hbm, acc.at[pl.ds(sid * STRIPE, STRIPE)])
    plsc.subcore_barrier()

    row_base0 = wid * (PER_W // EB)

    @pl.loop(0, N_ITER)
    def _(it):
        row_base = row_base0 + it * SUB
        pltpu.sync_copy(src_hbm.at[pl.ds(row_base, SUB)], srcb)
        pltpu.sync_copy(dst_hbm.at[pl.ds(row_base, SUB)], dstb)
        pltpu.sync_copy(ea_hbm.at[pl.ds(row_base, SUB)], eab)
        cps = [pltpu.async_copy(q_hbm.at[srcb.at[j]], rows.at[j], gsem)
               for j in range(SUB)]
        for cp in cps:
            cp.wait()

        for j in range(SUB):
            @pl.loop(0, EB, step=16)
            def _(e):
                ea16 = eab[j, pl.ds(e, 16)]
                for k in range(16):
                    s = ea16[k]
                    rows[j, e + k, pl.ds(0, 16)] = rows[j, e + k, pl.ds(0, 16)] * s
                    rows[j, e + k, pl.ds(16, 16)] = rows[j, e + k, pl.ds(16, 16)] * s

        cps2 = [pltpu.async_copy(rows.at[j], acc.at[dstb.at[j]], ssem,
                                 add=True) for j in range(SUB)]
        for cp in cps2:
            cp.wait()

    plsc.subcore_barrier()
    pltpu.sync_copy(acc.at[pl.ds(sid * STRIPE, STRIPE)],
                    out_hbm.at[cid].at[pl.ds(sid * STRIPE, STRIPE)])


@functools.partial(
    pl.kernel,
    out_type=jax.ShapeDtypeStruct((NC, N_PAD), jnp.float32),
    mesh=_mesh,
    compiler_params=_sc_params,
    scratch_types=[
        pltpu.VMEM_SHARED((N_PAD,), jnp.float32),
        pltpu.VMEM((SUB, EB), jnp.int32),
        pltpu.VMEM((SUB, EB), jnp.int32),
        pltpu.VMEM((SUB, EB), jnp.float32),
        pltpu.VMEM((SUB, EB), jnp.float32),
        pltpu.SemaphoreType.DMA,
        pltpu.SemaphoreType.DMA,
    ],
)
def _sc_pass2(p_hbm, src_hbm, dst_hbm, ea_hbm, zero_hbm, out_hbm,
              acc, srcb, dstb, eab, vals, gsem, ssem):
    cid = lax.axis_index("c")
    sid = lax.axis_index("s")
    wid = cid * NS + sid

    pltpu.sync_copy(zero_hbm, acc.at[pl.ds(sid * STRIPE, STRIPE)])
    plsc.subcore_barrier()

    row_base0 = wid * (PER_W // EB)

    @pl.loop(0, N_ITER)
    def _(it):
        row_base = row_base0 + it * SUB
        pltpu.sync_copy(src_hbm.at[pl.ds(row_base, SUB)], srcb)
        pltpu.sync_copy(dst_hbm.at[pl.ds(row_base, SUB)], dstb)
        pltpu.sync_copy(ea_hbm.at[pl.ds(row_base, SUB)], eab)
        cps = [pltpu.async_copy(p_hbm.at[srcb.at[j]], vals.at[j], gsem)
               for j in range(SUB)]
        for cp in cps:
            cp.wait()

        for j in range(SUB):
            @pl.loop(0, EB, step=16)
            def _(e):
                vals[j, pl.ds(e, 16)] = (vals[j, pl.ds(e, 16)]
                                         * eab[j, pl.ds(e, 16)])

        cps2 = [pltpu.async_copy(vals.at[j], acc.at[dstb.at[j]], ssem,
                                 add=True) for j in range(SUB)]
        for cp in cps2:
            cp.wait()

    plsc.subcore_barrier()
    pltpu.sync_copy(acc.at[pl.ds(sid * STRIPE, STRIPE)],
                    out_hbm.at[cid].at[pl.ds(sid * STRIPE, STRIPE)])


_BLK = 2000
_GRID = N // _BLK


def _tc1_body(x_ref, wn1_ref, ws1_ref, bs1_ref, q_ref, xs_ref):
    xb = x_ref[...]
    q_ref[...] = jnp.dot(xb, wn1_ref[...], preferred_element_type=jnp.float32, precision=lax.Precision.HIGHEST)
    xs_ref[...] = (jnp.dot(xb, ws1_ref[...], preferred_element_type=jnp.float32, precision=lax.Precision.HIGHEST)
                   + bs1_ref[...])


def _tc2_body(xs_ref, r0_ref, r1_ref, wit_ref, wgt_ref, wot_ref,
              bi_ref, bg_ref, bo_ref, ws2_ref, wn2_ref, wl1_ref, wl2_ref,
              bs2_ref, bl1_ref, bl2_ref,
              h_ref, c_ref, p_ref, s_ref):
    f32 = jnp.float32
    x1 = jax.nn.relu(xs_ref[...] + r0_ref[...] + r1_ref[...])
    i = jax.nn.sigmoid(jnp.dot(x1, wit_ref[...], preferred_element_type=f32, precision=lax.Precision.HIGHEST)
                       + bi_ref[...])
    g = jnp.tanh(jnp.dot(x1, wgt_ref[...], preferred_element_type=f32, precision=lax.Precision.HIGHEST)
                 + bg_ref[...])
    o = jax.nn.sigmoid(jnp.dot(x1, wot_ref[...], preferred_element_type=f32, precision=lax.Precision.HIGHEST)
                       + bo_ref[...])
    c = i * g
    h = o * jnp.tanh(c)
    c_ref[...] = c
    h_ref[...] = h
    wf = jnp.dot(wl1_ref[...], wl2_ref[...], preferred_element_type=f32, precision=lax.Precision.HIGHEST)
    av = jnp.dot(ws2_ref[...], wf, preferred_element_type=f32, precision=lax.Precision.HIGHEST)
    bv = jnp.dot(wn2_ref[...], wf, preferred_element_type=f32, precision=lax.Precision.HIGHEST)
    const = (jnp.dot(bs2_ref[...], wf, preferred_element_type=f32, precision=lax.Precision.HIGHEST)
             + jnp.dot(bl1_ref[...], wl2_ref[...], preferred_element_type=f32, precision=lax.Precision.HIGHEST)
             + bl2_ref[...])
    p_ref[...] = (jnp.dot(x1, bv[:D], preferred_element_type=f32, precision=lax.Precision.HIGHEST)
                  + jnp.dot(h, bv[D:], preferred_element_type=f32, precision=lax.Precision.HIGHEST))
    s_ref[...] = (jnp.dot(x1, av[:D], preferred_element_type=f32, precision=lax.Precision.HIGHEST)
                  + jnp.dot(h, av[D:], preferred_element_type=f32, precision=lax.Precision.HIGHEST) + const)


def _tc3_body(r0_ref, r1_ref, s_ref, z_ref):
    z_ref[...] = r0_ref[...] + r1_ref[...] + s_ref[...]


def kernel(x, edge_index, edge_attr, Ws1, Wn1, bs1, Wi, bi, Wf, bf, Wg, bg,
           Wo, bo, Ws2, Wn2, bs2, Wl1, bl1, Wl2, bl2):
    f32 = jnp.float32
    src = edge_index[0]
    dst = edge_index[1]

    n_extra = E_PAD - E
    pad_idx = (jnp.arange(n_extra, dtype=jnp.int32) * 61) % N
    src_p = jnp.concatenate([src.astype(jnp.int32), pad_idx]).reshape(-1, EB)
    dst_p = jnp.concatenate([dst.astype(jnp.int32), pad_idx]).reshape(-1, EB)
    ea_p = jnp.concatenate([edge_attr,
                            jnp.zeros((n_extra,), f32)]).reshape(-1, EB)

    zero2d = jnp.zeros((STRIPE, D), f32)
    zero1d = jnp.zeros((STRIPE,), f32)

    wspec = pl.BlockSpec((D, D), lambda i: (0, 0))
    bspec = pl.BlockSpec((1, D), lambda i: (0, 0))
    nspec = pl.BlockSpec((_BLK, D), lambda i: (i, 0))
    q, xs = pl.pallas_call(
        _tc1_body,
        grid=(_GRID,),
        in_specs=[nspec, wspec, wspec, bspec],
        out_specs=[nspec, nspec],
        out_shape=[jax.ShapeDtypeStruct((N, D), f32)] * 2,
    )(x, Wn1, Ws1, bs1.reshape(1, D))

    parts1 = _sc_pass1(q, src_p, dst_p, ea_p, zero2d)
    r0 = parts1[0, :N, :]
    r1 = parts1[1, :N, :]

    wspec64 = pl.BlockSpec((2 * D, 2 * D), lambda i: (0, 0))
    pspec = pl.BlockSpec((_BLK, 1), lambda i: (i, 0))
    h, c, p, s = pl.pallas_call(
        _tc2_body,
        grid=(_GRID,),
        in_specs=[nspec, nspec, nspec, wspec, wspec, wspec,
                  bspec, bspec, bspec, wspec64, wspec64,
                  pl.BlockSpec((2 * D, D), lambda i: (0, 0)),
                  pl.BlockSpec((D, 1), lambda i: (0, 0)),
                  pl.BlockSpec((1, 2 * D), lambda i: (0, 0)),
                  bspec, pl.BlockSpec((1, 1), lambda i: (0, 0))],
        out_specs=[nspec, nspec, pspec, pspec],
        out_shape=[jax.ShapeDtypeStruct((N, D), f32),
                   jax.ShapeDtypeStruct((N, D), f32),
                   jax.ShapeDtypeStruct((N, 1), f32),
                   jax.ShapeDtypeStruct((N, 1), f32)],
    )(xs, r0, r1, Wi[:D], Wg[:D], Wo[:D],
      bi.reshape(1, D), bg.reshape(1, D), bo.reshape(1, D), Ws2, Wn2,
      Wl1, Wl2, bs2.reshape(1, 2 * D), bl1.reshape(1, D), bl2.reshape(1, 1))

    p_flat = jnp.pad(p.reshape(N), (0, N_PAD - N))
    parts2 = _sc_pass2(p_flat, src_p, dst_p, ea_p, zero1d)

    z = pl.pallas_call(
        _tc3_body,
        grid=(1,),
        in_specs=[pl.BlockSpec((N,), lambda i: (0,))] * 3,
        out_specs=pl.BlockSpec((N,), lambda i: (0,)),
        out_shape=jax.ShapeDtypeStruct((N,), f32),
    )(parts2[0, :N], parts2[1, :N], s.reshape(N))

    return (z.reshape(N, 1), h, c)
